# baseline (speedup 1.0000x reference)
import math
import functools

import jax
import jax.numpy as jnp
from jax import lax
from jax.experimental import pallas as pl
from jax.experimental.pallas import tpu as pltpu
from jax.experimental.pallas import tpu_sc as plsc

D = 768
H = 12
HD = 64
Q = 256
K = 4096
KTU = max(32, min(int(math.ceil(0.1 * K)), K))
SCALE = HD ** -0.5

_ONE_BITS = 0x3F800000


def _seg_matrix():
    lane = jax.lax.broadcasted_iota(jnp.int32, (D, H), 0)
    head = jax.lax.broadcasted_iota(jnp.int32, (D, H), 1)
    return (lane // HD == head).astype(jnp.float32)


def _ln_norm_proj(x, g, b, W, bias):
    m = jnp.mean(x, axis=-1, keepdims=True)
    v = jnp.mean((x - m) ** 2, axis=-1, keepdims=True)
    xn = (x - m) * jax.lax.rsqrt(v + 1e-5) * g + b
    return jnp.dot(xn, W, preferred_element_type=jnp.float32) + bias


def _headwise_l2norm(x, seg):
    ssum = jnp.dot(x * x, seg, preferred_element_type=jnp.float32)
    nrm = jnp.maximum(jnp.sqrt(ssum), 1e-12)
    inv_full = jnp.dot(1.0 / nrm, seg.T, preferred_element_type=jnp.float32)
    return x * inv_full


def _stage_aq_kernel(query_ref, qpos_ref,
                     q_ln_g_ref, q_ln_b_ref, q_W_ref, q_b_ref,
                     q4n_ref, qp_ref):
    seg = _seg_matrix()
    q = query_ref[...] + qpos_ref[...]
    qp = _ln_norm_proj(q, q_ln_g_ref[...], q_ln_b_ref[...],
                       q_W_ref[...], q_b_ref[...])
    qp_ref[...] = qp
    q4n_ref[...] = _headwise_l2norm(qp, seg)


def _stage_ak_kernel(key_ref, kpos_ref,
                     k_ln_g_ref, k_ln_b_ref, k_W_ref, k_b_ref,
                     k4n_ref, v_ref):
    seg = _seg_matrix()
    kk = key_ref[...] + kpos_ref[...]
    v_ref[...] = kk
    kp = _ln_norm_proj(kk, k_ln_g_ref[...], k_ln_b_ref[...],
                       k_W_ref[...], k_b_ref[...])
    k4n_ref[...] = _headwise_l2norm(kp, seg)


def _encode(x):
    i = jax.lax.bitcast_convert_type(x, jnp.int32)
    return i ^ jax.lax.shift_right_logical(
        jax.lax.shift_right_arithmetic(i, 31), 1)


N_ITERS = 24


def _attend(q, k, v):
    sim = jax.lax.dot_general(q, k, (((1,), (1,)), ((), ())),
                              preferred_element_type=jnp.float32)
    enc = _encode(sim)

    def body(_, carry):
        lo, hi = carry
        mid = lo + jax.lax.shift_right_arithmetic(hi - lo, 1)
        cnt = jnp.sum((enc >= mid).astype(jnp.int32), axis=-1, keepdims=True)
        ge = cnt >= KTU
        return jnp.where(ge, mid, lo), jnp.where(ge, hi, mid)

    lo0 = jnp.full((Q, 1), -(_ONE_BITS + 1), jnp.int32)
    hi0 = jnp.full((Q, 1), _ONE_BITS + 1, jnp.int32)
    lo, _ = jax.lax.fori_loop(0, N_ITERS, body, (lo0, hi0))

    p = jnp.where(enc >= lo, jnp.exp(sim * SCALE), 0.0)
    attn = p / jnp.sum(p, axis=-1, keepdims=True)
    return jnp.dot(attn, v, preferred_element_type=jnp.float32)


def _stage_b_kernel(q_ref, k_ref, v_ref, out_ref):
    for h in range(2):
        sl = slice(h * HD, (h + 1) * HD)
        out_ref[:, sl] = _attend(q_ref[:, sl], k_ref[:, sl], v_ref[:, sl])


_R = _ONE_BITS + 1
NC = 2
NS = 16
NW = NC * NS
ROWS = H * Q
RPW = ROWS // NW


def _decode(e):
    mask = jax.lax.shift_right_logical(
        jax.lax.shift_right_arithmetic(e, 31), 1)
    return jax.lax.bitcast_convert_type(e ^ mask, jnp.float32)


def _stage_b1_kernel(q_ref, k_ref, out_ref):
    for h in range(2):
        sl = slice(h * HD, (h + 1) * HD)
        sim = jax.lax.dot_general(q_ref[:, sl], k_ref[:, sl],
                                  (((1,), (1,)), ((), ())),
                                  preferred_element_type=jnp.float32)
        out_ref[h * Q:(h + 1) * Q, :] = _encode(sim)


def _scan_top(hist, need, iota):
    def chunk(i, carry):
        total, found, bsel, above = carry
        c = 15 - i
        hv = hist[pl.ds(c * 16, 16)]
        rv = lax.rev(hv, (0,))
        sfx = plsc.cumsum(rv) + total
        hit = sfx >= need
        cand = jnp.where(hit, 15 - iota, -1)
        bmax = jnp.max(cand)
        anyhit = bmax >= 0
        b_here = c * 16 + bmax
        sel = jnp.where(iota == (15 - bmax), sfx - rv, 0)
        above_here = jnp.sum(sel)
        take = jnp.logical_and(found == 0, anyhit)
        bsel = jnp.where(take, b_here, bsel)
        above = jnp.where(take, above_here, above)
        found = jnp.where(anyhit, jnp.int32(1), found)
        total = total + jnp.sum(hv)
        return total, found, bsel, above

    init = (jnp.int32(0), jnp.int32(0), jnp.int32(0), jnp.int32(0))
    _, _, bsel, above = lax.fori_loop(0, 16, chunk, init)
    return bsel, above


def _sc_select_kernel(enc_hbm, out_hbm, rowbuf, h1, h2, res):
    wid = lax.axis_index("s") * NC + lax.axis_index("c")
    base = wid * RPW
    ones = jnp.ones((16,), jnp.int32)
    iota = lax.iota(jnp.int32, 16)

    def grp_body(g, _):
        def row_body(j, vec):
            r = g * 16 + j
            pltpu.sync_copy(enc_hbm.at[base + r], rowbuf)

            def z(i, _):
                h1[pl.ds(i * 16, 16)] = jnp.zeros((16,), jnp.int32)
                h2[pl.ds(i * 16, 16)] = jnp.zeros((16,), jnp.int32)
                return 0
            lax.fori_loop(0, 16, z, 0)

            def s1(i, _):
                v = rowbuf[pl.ds(i * 16, 16)]
                b = jax.lax.shift_right_arithmetic(v + _R, 23)
                plsc.addupdate_scatter(h1, [b], ones)
                return 0
            lax.fori_loop(0, K // 16, s1, 0)

            b1, above1 = _scan_top(h1, jnp.int32(KTU), iota)

            def s2(i, _):
                v = rowbuf[pl.ds(i * 16, 16)]
                u = v + _R
                m = jax.lax.shift_right_arithmetic(u, 23) == b1
                b = jax.lax.shift_right_arithmetic(u, 15) & 0xFF
                plsc.addupdate_scatter(h2, [b], ones, mask=m)
                return 0
            lax.fori_loop(0, K // 16, s2, 0)

            b2, _ = _scan_top(h2, jnp.int32(KTU) - above1, iota)
            t = jax.lax.shift_left(jax.lax.shift_left(b1, 8) | b2, 15) - _R
            return jnp.where(iota == j, t, vec)

        vec = lax.fori_loop(0, 16, row_body, jnp.zeros((16,), jnp.int32))
        res[pl.ds(g * 16, 16)] = vec
        return 0

    lax.fori_loop(0, RPW // 16, grp_body, 0)
    pltpu.sync_copy(res, out_hbm.at[pl.ds(base, RPW)])


def _sc_select(enc_all):
    fn = pl.kernel(
        _sc_select_kernel,
        out_type=jax.ShapeDtypeStruct((ROWS,), jnp.int32),
        mesh=plsc.VectorSubcoreMesh(core_axis_name="c", subcore_axis_name="s"),
        compiler_params=pltpu.CompilerParams(needs_layout_passes=False),
        scratch_types=[
            pltpu.VMEM((K,), jnp.int32),
            pltpu.VMEM((256,), jnp.int32),
            pltpu.VMEM((256,), jnp.int32),
            pltpu.VMEM((RPW,), jnp.int32),
        ],
    )
    return fn(enc_all)


def _stage_b2_kernel(enc_ref, t_ref, v_ref, out_ref):
    t = t_ref[...]
    for h in range(2):
        enc = enc_ref[h * Q:(h + 1) * Q, :]
        sim = _decode(enc)
        p = jnp.where(enc >= t[h * Q:(h + 1) * Q], jnp.exp(sim * SCALE), 0.0)
        attn = p / jnp.sum(p, axis=-1, keepdims=True)
        sl = slice(h * HD, (h + 1) * HD)
        out_ref[:, sl] = jnp.dot(attn, v_ref[:, sl],
                                 preferred_element_type=jnp.float32)


def _stage_c_kernel(merge_ref, qp_ref, residual_ref,
                    p_W_ref, p_b_ref, f_W_ref, f_b_ref, alpha_ref, out_ref):
    merge = merge_ref[...]
    inter = jnp.dot(merge * qp_ref[...], p_W_ref[...],
                    preferred_element_type=jnp.float32) + p_b_ref[...]
    n2 = jnp.sum(inter * inter, axis=0, keepdims=True)
    nrm = jnp.maximum(jnp.sqrt(n2), 1e-12)
    out = inter / nrm * alpha_ref[...] + merge
    out = jnp.dot(out, f_W_ref[...],
                  preferred_element_type=jnp.float32) + f_b_ref[...]
    out_ref[...] = residual_ref[...] + out


def kernel(query, key_t, query_pos, key_pos, q_ln_g, q_ln_b, q_W, q_b,
           k_ln_g, k_ln_b, k_W, k_b, p_W, p_b, f_W, f_b, alpha):
    q2 = query[:, 0, :]
    qp2 = query_pos[:, 0, :]
    k2 = key_t[:, 0, :]
    kp2 = key_pos[:, 0, :]

    q4n, qp = pl.pallas_call(
        _stage_aq_kernel,
        out_shape=[
            jax.ShapeDtypeStruct((Q, D), jnp.float32),
            jax.ShapeDtypeStruct((Q, D), jnp.float32),
        ],
    )(q2, qp2, q_ln_g, q_ln_b, q_W, q_b)

    KB = 1024
    k4n, v = pl.pallas_call(
        _stage_ak_kernel,
        grid=(K // KB,),
        in_specs=[
            pl.BlockSpec((KB, D), lambda i: (i, 0)),
            pl.BlockSpec((KB, D), lambda i: (i, 0)),
            pl.BlockSpec((D,), lambda i: (0,)),
            pl.BlockSpec((D,), lambda i: (0,)),
            pl.BlockSpec((D, D), lambda i: (0, 0)),
            pl.BlockSpec((D,), lambda i: (0,)),
        ],
        out_specs=[
            pl.BlockSpec((KB, D), lambda i: (i, 0)),
            pl.BlockSpec((KB, D), lambda i: (i, 0)),
        ],
        out_shape=[
            jax.ShapeDtypeStruct((K, D), jnp.float32),
            jax.ShapeDtypeStruct((K, D), jnp.float32),
        ],
    )(k2, kp2, k_ln_g, k_ln_b, k_W, k_b)

    enc_all = pl.pallas_call(
        _stage_b1_kernel,
        grid=(H // 2,),
        in_specs=[
            pl.BlockSpec((Q, 2 * HD), lambda h: (0, h)),
            pl.BlockSpec((K, 2 * HD), lambda h: (0, h)),
        ],
        out_specs=pl.BlockSpec((2 * Q, K), lambda h: (h, 0)),
        out_shape=jax.ShapeDtypeStruct((ROWS, K), jnp.int32),
    )(q4n, k4n)

    thresh = _sc_select(enc_all).reshape(ROWS, 1)

    merge = pl.pallas_call(
        _stage_b2_kernel,
        grid=(H // 2,),
        in_specs=[
            pl.BlockSpec((2 * Q, K), lambda h: (h, 0)),
            pl.BlockSpec((2 * Q, 1), lambda h: (h, 0)),
            pl.BlockSpec((K, 2 * HD), lambda h: (0, h)),
        ],
        out_specs=pl.BlockSpec((Q, 2 * HD), lambda h: (0, h)),
        out_shape=jax.ShapeDtypeStruct((Q, D), jnp.float32),
    )(enc_all, thresh, v)

    out = pl.pallas_call(
        _stage_c_kernel,
        out_shape=jax.ShapeDtypeStruct((Q, D), jnp.float32),
    )(merge, qp, q2, p_W, p_b, f_W, f_b, alpha[0])

    return out[:, None, :]


# speedup vs baseline: 1.1897x; 1.1897x over previous
import math
import functools

import jax
import jax.numpy as jnp
from jax import lax
from jax.experimental import pallas as pl
from jax.experimental.pallas import tpu as pltpu
from jax.experimental.pallas import tpu_sc as plsc

D = 768
H = 12
HD = 64
Q = 256
K = 4096
KTU = max(32, min(int(math.ceil(0.1 * K)), K))
SCALE = HD ** -0.5

_ONE_BITS = 0x3F800000


def _seg_matrix():
    lane = jax.lax.broadcasted_iota(jnp.int32, (D, H), 0)
    head = jax.lax.broadcasted_iota(jnp.int32, (D, H), 1)
    return (lane // HD == head).astype(jnp.float32)


def _ln_norm_proj(x, g, b, W, bias):
    m = jnp.mean(x, axis=-1, keepdims=True)
    v = jnp.mean((x - m) ** 2, axis=-1, keepdims=True)
    xn = (x - m) * jax.lax.rsqrt(v + 1e-5) * g + b
    return jnp.dot(xn, W, preferred_element_type=jnp.float32) + bias


def _headwise_l2norm(x, seg):
    ssum = jnp.dot(x * x, seg, preferred_element_type=jnp.float32)
    nrm = jnp.maximum(jnp.sqrt(ssum), 1e-12)
    inv_full = jnp.dot(1.0 / nrm, seg.T, preferred_element_type=jnp.float32)
    return x * inv_full


def _stage_aq_kernel(query_ref, qpos_ref,
                     q_ln_g_ref, q_ln_b_ref, q_W_ref, q_b_ref,
                     q4n_ref, qp_ref):
    seg = _seg_matrix()
    q = query_ref[...] + qpos_ref[...]
    qp = _ln_norm_proj(q, q_ln_g_ref[...], q_ln_b_ref[...],
                       q_W_ref[...], q_b_ref[...])
    qp_ref[...] = qp
    q4n_ref[...] = _headwise_l2norm(qp, seg)


def _stage_ak_kernel(key_ref, kpos_ref,
                     k_ln_g_ref, k_ln_b_ref, k_W_ref, k_b_ref,
                     k4n_ref, v_ref):
    seg = _seg_matrix()
    kk = key_ref[...] + kpos_ref[...]
    v_ref[...] = kk
    kp = _ln_norm_proj(kk, k_ln_g_ref[...], k_ln_b_ref[...],
                       k_W_ref[...], k_b_ref[...])
    k4n_ref[...] = _headwise_l2norm(kp, seg)


def _encode(x):
    i = jax.lax.bitcast_convert_type(x, jnp.int32)
    return i ^ jax.lax.shift_right_logical(
        jax.lax.shift_right_arithmetic(i, 31), 1)


N_ITERS = 24


def _attend(q, k, v):
    sim = jax.lax.dot_general(q, k, (((1,), (1,)), ((), ())),
                              preferred_element_type=jnp.float32)
    enc = _encode(sim)

    def body(_, carry):
        lo, hi = carry
        mid = lo + jax.lax.shift_right_arithmetic(hi - lo, 1)
        cnt = jnp.sum((enc >= mid).astype(jnp.int32), axis=-1, keepdims=True)
        ge = cnt >= KTU
        return jnp.where(ge, mid, lo), jnp.where(ge, hi, mid)

    lo0 = jnp.full((Q, 1), -(_ONE_BITS + 1), jnp.int32)
    hi0 = jnp.full((Q, 1), _ONE_BITS + 1, jnp.int32)
    lo, _ = jax.lax.fori_loop(0, N_ITERS, body, (lo0, hi0))

    p = jnp.where(enc >= lo, jnp.exp(sim * SCALE), 0.0)
    attn = p / jnp.sum(p, axis=-1, keepdims=True)
    return jnp.dot(attn, v, preferred_element_type=jnp.float32)


def _stage_b_kernel(q_ref, k_ref, v_ref, out_ref):
    for h in range(2):
        sl = slice(h * HD, (h + 1) * HD)
        out_ref[:, sl] = _attend(q_ref[:, sl], k_ref[:, sl], v_ref[:, sl])


_R = _ONE_BITS + 1
NC = 2
NS = 16
NW = NC * NS
ROWS = H * Q
RPW = ROWS // NW


def _decode(e):
    mask = jax.lax.shift_right_logical(
        jax.lax.shift_right_arithmetic(e, 31), 1)
    return jax.lax.bitcast_convert_type(e ^ mask, jnp.float32)


def _stage_b1_kernel(q_ref, k_ref, out_ref):
    for h in range(2):
        sl = slice(h * HD, (h + 1) * HD)
        sim = jax.lax.dot_general(q_ref[:, sl], k_ref[:, sl],
                                  (((1,), (1,)), ((), ())),
                                  preferred_element_type=jnp.float32)
        out_ref[h * Q:(h + 1) * Q, :] = _encode(sim)


def _scan_top(hist, need, iota):
    def chunk(i, carry):
        total, found, bsel, above = carry
        c = 15 - i
        hv = hist[pl.ds(c * 16, 16)]
        rv = lax.rev(hv, (0,))
        sfx = plsc.cumsum(rv) + total
        hit = sfx >= need
        cand = jnp.where(hit, 15 - iota, -1)
        bmax = jnp.max(cand)
        anyhit = bmax >= 0
        b_here = c * 16 + bmax
        sel = jnp.where(iota == (15 - bmax), sfx - rv, 0)
        above_here = jnp.sum(sel)
        take = jnp.logical_and(found == 0, anyhit)
        bsel = jnp.where(take, b_here, bsel)
        above = jnp.where(take, above_here, above)
        found = jnp.where(anyhit, jnp.int32(1), found)
        total = total + jnp.sum(hv)
        return total, found, bsel, above

    init = (jnp.int32(0), jnp.int32(0), jnp.int32(0), jnp.int32(0))
    _, _, bsel, above = lax.fori_loop(0, 16, chunk, init)
    return bsel, above


_UNROLL = 8


def _sc_select_kernel(enc_hbm, out_hbm, rowbuf0, rowbuf1, h1, h2, res,
                      sem0, sem1):
    wid = lax.axis_index("s") * NC + lax.axis_index("c")
    base = wid * RPW
    ones = jnp.ones((16,), jnp.int32)
    iota = lax.iota(jnp.int32, 16)
    sems = (sem0, sem1)
    bufs = (rowbuf0, rowbuf1)

    def _copy(row, b, sem):
        return pltpu.make_async_copy(enc_hbm.at[row], bufs[b], sem)

    def _process(b, r, vec):
        buf = bufs[b]
        for i in range(16):
            h1[pl.ds(i * 16, 16)] = jnp.zeros((16,), jnp.int32)
            h2[pl.ds(i * 16, 16)] = jnp.zeros((16,), jnp.int32)

        def s1(i, _):
            for u in range(_UNROLL):
                v = buf[pl.ds((i * _UNROLL + u) * 16, 16)]
                bk = jax.lax.shift_right_arithmetic(v + _R, 23)
                plsc.addupdate_scatter(h1, [bk], ones)
            return 0
        lax.fori_loop(0, K // 16 // _UNROLL, s1, 0)

        b1, above1 = _scan_top(h1, jnp.int32(KTU), iota)

        def s2(i, _):
            for u in range(_UNROLL):
                v = buf[pl.ds((i * _UNROLL + u) * 16, 16)]
                uu = v + _R
                m = jax.lax.shift_right_arithmetic(uu, 23) == b1
                bk = jax.lax.shift_right_arithmetic(uu, 15) & 0xFF
                plsc.addupdate_scatter(h2, [bk], ones, mask=m)
            return 0
        lax.fori_loop(0, K // 16 // _UNROLL, s2, 0)

        b2, _ = _scan_top(h2, jnp.int32(KTU) - above1, iota)
        t = jax.lax.shift_left(jax.lax.shift_left(b1, 8) | b2, 15) - _R
        return jnp.where(iota == (r % 16), t, vec)

    _copy(base, 0, sems[0]).start()

    def outer(i, vec):
        for b in range(2):
            r = i * 2 + b
            row = base + r
            _copy(row, b, sems[b]).wait()
            nxt = jnp.minimum(row + 2, ROWS - 1)
            nb = 1 - b
            if b == 0:
                _copy(base + r + 1, nb, sems[nb]).start()
            else:
                _copy(nxt, nb, sems[nb]).start()
            vec = _process(b, r, vec)

            @pl.when(r % 16 == 15)
            def _():
                res[pl.ds((r // 16) * 16, 16)] = vec
        return vec

    lax.fori_loop(0, RPW // 2, outer, jnp.zeros((16,), jnp.int32))
    _copy(base, 0, sems[0]).wait()
    pltpu.sync_copy(res, out_hbm.at[pl.ds(base, RPW)])


def _sc_select(enc_all):
    fn = pl.kernel(
        _sc_select_kernel,
        out_type=jax.ShapeDtypeStruct((ROWS,), jnp.int32),
        mesh=plsc.VectorSubcoreMesh(core_axis_name="c", subcore_axis_name="s"),
        compiler_params=pltpu.CompilerParams(needs_layout_passes=False),
        scratch_types=[
            pltpu.VMEM((K,), jnp.int32),
            pltpu.VMEM((K,), jnp.int32),
            pltpu.VMEM((256,), jnp.int32),
            pltpu.VMEM((256,), jnp.int32),
            pltpu.VMEM((RPW,), jnp.int32),
            pltpu.SemaphoreType.DMA,
            pltpu.SemaphoreType.DMA,
        ],
    )
    return fn(enc_all)


def _stage_b2_kernel(enc_ref, t_ref, v_ref, out_ref):
    t = t_ref[...]
    for h in range(2):
        enc = enc_ref[h * Q:(h + 1) * Q, :]
        sim = _decode(enc)
        p = jnp.where(enc >= t[h * Q:(h + 1) * Q], jnp.exp(sim * SCALE), 0.0)
        attn = p / jnp.sum(p, axis=-1, keepdims=True)
        sl = slice(h * HD, (h + 1) * HD)
        out_ref[:, sl] = jnp.dot(attn, v_ref[:, sl],
                                 preferred_element_type=jnp.float32)


def _stage_c_kernel(merge_ref, qp_ref, residual_ref,
                    p_W_ref, p_b_ref, f_W_ref, f_b_ref, alpha_ref, out_ref):
    merge = merge_ref[...]
    inter = jnp.dot(merge * qp_ref[...], p_W_ref[...],
                    preferred_element_type=jnp.float32) + p_b_ref[...]
    n2 = jnp.sum(inter * inter, axis=0, keepdims=True)
    nrm = jnp.maximum(jnp.sqrt(n2), 1e-12)
    out = inter / nrm * alpha_ref[...] + merge
    out = jnp.dot(out, f_W_ref[...],
                  preferred_element_type=jnp.float32) + f_b_ref[...]
    out_ref[...] = residual_ref[...] + out


def kernel(query, key_t, query_pos, key_pos, q_ln_g, q_ln_b, q_W, q_b,
           k_ln_g, k_ln_b, k_W, k_b, p_W, p_b, f_W, f_b, alpha):
    q2 = query[:, 0, :]
    qp2 = query_pos[:, 0, :]
    k2 = key_t[:, 0, :]
    kp2 = key_pos[:, 0, :]

    q4n, qp = pl.pallas_call(
        _stage_aq_kernel,
        out_shape=[
            jax.ShapeDtypeStruct((Q, D), jnp.float32),
            jax.ShapeDtypeStruct((Q, D), jnp.float32),
        ],
    )(q2, qp2, q_ln_g, q_ln_b, q_W, q_b)

    KB = 1024
    k4n, v = pl.pallas_call(
        _stage_ak_kernel,
        grid=(K // KB,),
        in_specs=[
            pl.BlockSpec((KB, D), lambda i: (i, 0)),
            pl.BlockSpec((KB, D), lambda i: (i, 0)),
            pl.BlockSpec((D,), lambda i: (0,)),
            pl.BlockSpec((D,), lambda i: (0,)),
            pl.BlockSpec((D, D), lambda i: (0, 0)),
            pl.BlockSpec((D,), lambda i: (0,)),
        ],
        out_specs=[
            pl.BlockSpec((KB, D), lambda i: (i, 0)),
            pl.BlockSpec((KB, D), lambda i: (i, 0)),
        ],
        out_shape=[
            jax.ShapeDtypeStruct((K, D), jnp.float32),
            jax.ShapeDtypeStruct((K, D), jnp.float32),
        ],
    )(k2, kp2, k_ln_g, k_ln_b, k_W, k_b)

    enc_all = pl.pallas_call(
        _stage_b1_kernel,
        grid=(H // 2,),
        in_specs=[
            pl.BlockSpec((Q, 2 * HD), lambda h: (0, h)),
            pl.BlockSpec((K, 2 * HD), lambda h: (0, h)),
        ],
        out_specs=pl.BlockSpec((2 * Q, K), lambda h: (h, 0)),
        out_shape=jax.ShapeDtypeStruct((ROWS, K), jnp.int32),
    )(q4n, k4n)

    thresh = _sc_select(enc_all).reshape(ROWS, 1)

    merge = pl.pallas_call(
        _stage_b2_kernel,
        grid=(H // 2,),
        in_specs=[
            pl.BlockSpec((2 * Q, K), lambda h: (h, 0)),
            pl.BlockSpec((2 * Q, 1), lambda h: (h, 0)),
            pl.BlockSpec((K, 2 * HD), lambda h: (0, h)),
        ],
        out_specs=pl.BlockSpec((Q, 2 * HD), lambda h: (0, h)),
        out_shape=jax.ShapeDtypeStruct((Q, D), jnp.float32),
    )(enc_all, thresh, v)

    out = pl.pallas_call(
        _stage_c_kernel,
        out_shape=jax.ShapeDtypeStruct((Q, D), jnp.float32),
    )(merge, qp, q2, p_W, p_b, f_W, f_b, alpha[0])

    return out[:, None, :]


# speedup vs baseline: 2.1261x; 1.7871x over previous
import math
import functools

import jax
import jax.numpy as jnp
from jax.experimental import pallas as pl
from jax.experimental.pallas import tpu as pltpu

D = 768
H = 12
HD = 64
Q = 256
K = 4096
KTU = max(32, min(int(math.ceil(0.1 * K)), K))
SCALE = HD ** -0.5

_ONE_BITS = 0x3F800000


def _seg_matrix():
    lane = jax.lax.broadcasted_iota(jnp.int32, (D, H), 0)
    head = jax.lax.broadcasted_iota(jnp.int32, (D, H), 1)
    return (lane // HD == head).astype(jnp.float32)


def _ln_norm_proj(x, g, b, W, bias):
    m = jnp.mean(x, axis=-1, keepdims=True)
    v = jnp.mean((x - m) ** 2, axis=-1, keepdims=True)
    xn = (x - m) * jax.lax.rsqrt(v + 1e-5) * g + b
    return jnp.dot(xn, W, preferred_element_type=jnp.float32) + bias


def _headwise_l2norm(x, seg):
    ssum = jnp.dot(x * x, seg, preferred_element_type=jnp.float32)
    nrm = jnp.maximum(jnp.sqrt(ssum), 1e-12)
    inv_full = jnp.dot(1.0 / nrm, seg.T, preferred_element_type=jnp.float32)
    return x * inv_full


def _stage_aq_kernel(query_ref, qpos_ref,
                     q_ln_g_ref, q_ln_b_ref, q_W_ref, q_b_ref,
                     q4n_ref, qp_ref):
    seg = _seg_matrix()
    q = query_ref[...] + qpos_ref[...]
    qp = _ln_norm_proj(q, q_ln_g_ref[...], q_ln_b_ref[...],
                       q_W_ref[...], q_b_ref[...])
    qp_ref[...] = qp
    q4n_ref[...] = _headwise_l2norm(qp, seg)


def _stage_ak_kernel(key_ref, kpos_ref,
                     k_ln_g_ref, k_ln_b_ref, k_W_ref, k_b_ref,
                     k4n_ref, v_ref):
    seg = _seg_matrix()
    kk = key_ref[...] + kpos_ref[...]
    v_ref[...] = kk
    kp = _ln_norm_proj(kk, k_ln_g_ref[...], k_ln_b_ref[...],
                       k_W_ref[...], k_b_ref[...])
    k4n_ref[...] = _headwise_l2norm(kp, seg)


def _encode(x):
    i = jax.lax.bitcast_convert_type(x, jnp.int32)
    return i ^ jax.lax.shift_right_logical(
        jax.lax.shift_right_arithmetic(i, 31), 1)


N_ITERS = 24


def _attend(q, k, v):
    sim = jax.lax.dot_general(q, k, (((1,), (1,)), ((), ())),
                              preferred_element_type=jnp.float32)
    enc = _encode(sim)
    ones_mat = jnp.ones((K, 8), jnp.float32)

    def body(_, carry):
        lo, hi = carry
        mid = lo + jax.lax.shift_right_arithmetic(hi - lo, 1)
        ind = (enc >= mid).astype(jnp.float32)
        cnt = jnp.dot(ind, ones_mat,
                      preferred_element_type=jnp.float32)[:, :1]
        ge = cnt >= float(KTU)
        return jnp.where(ge, mid, lo), jnp.where(ge, hi, mid)

    lo0 = jnp.full((Q, 1), -(_ONE_BITS + 1), jnp.int32)
    hi0 = jnp.full((Q, 1), _ONE_BITS + 1, jnp.int32)
    lo, _ = jax.lax.fori_loop(0, N_ITERS, body, (lo0, hi0))

    p = jnp.where(enc >= lo, jnp.exp(sim * SCALE), 0.0)
    attn = p / jnp.sum(p, axis=-1, keepdims=True)
    return jnp.dot(attn, v, preferred_element_type=jnp.float32)


def _stage_b_kernel(q_ref, k_ref, v_ref, out_ref):
    for h in range(2):
        sl = slice(h * HD, (h + 1) * HD)
        out_ref[:, sl] = _attend(q_ref[:, sl], k_ref[:, sl], v_ref[:, sl])


def _stage_c_kernel(merge_ref, qp_ref, residual_ref,
                    p_W_ref, p_b_ref, f_W_ref, f_b_ref, alpha_ref, out_ref):
    merge = merge_ref[...]
    inter = jnp.dot(merge * qp_ref[...], p_W_ref[...],
                    preferred_element_type=jnp.float32) + p_b_ref[...]
    n2 = jnp.sum(inter * inter, axis=0, keepdims=True)
    nrm = jnp.maximum(jnp.sqrt(n2), 1e-12)
    out = inter / nrm * alpha_ref[...] + merge
    out = jnp.dot(out, f_W_ref[...],
                  preferred_element_type=jnp.float32) + f_b_ref[...]
    out_ref[...] = residual_ref[...] + out


def kernel(query, key_t, query_pos, key_pos, q_ln_g, q_ln_b, q_W, q_b,
           k_ln_g, k_ln_b, k_W, k_b, p_W, p_b, f_W, f_b, alpha):
    q2 = query[:, 0, :]
    qp2 = query_pos[:, 0, :]
    k2 = key_t[:, 0, :]
    kp2 = key_pos[:, 0, :]

    q4n, qp = pl.pallas_call(
        _stage_aq_kernel,
        out_shape=[
            jax.ShapeDtypeStruct((Q, D), jnp.float32),
            jax.ShapeDtypeStruct((Q, D), jnp.float32),
        ],
    )(q2, qp2, q_ln_g, q_ln_b, q_W, q_b)

    KB = 1024
    k4n, v = pl.pallas_call(
        _stage_ak_kernel,
        grid=(K // KB,),
        in_specs=[
            pl.BlockSpec((KB, D), lambda i: (i, 0)),
            pl.BlockSpec((KB, D), lambda i: (i, 0)),
            pl.BlockSpec((D,), lambda i: (0,)),
            pl.BlockSpec((D,), lambda i: (0,)),
            pl.BlockSpec((D, D), lambda i: (0, 0)),
            pl.BlockSpec((D,), lambda i: (0,)),
        ],
        out_specs=[
            pl.BlockSpec((KB, D), lambda i: (i, 0)),
            pl.BlockSpec((KB, D), lambda i: (i, 0)),
        ],
        out_shape=[
            jax.ShapeDtypeStruct((K, D), jnp.float32),
            jax.ShapeDtypeStruct((K, D), jnp.float32),
        ],
    )(k2, kp2, k_ln_g, k_ln_b, k_W, k_b)

    merge = pl.pallas_call(
        _stage_b_kernel,
        grid=(H // 2,),
        in_specs=[
            pl.BlockSpec((Q, 2 * HD), lambda h: (0, h)),
            pl.BlockSpec((K, 2 * HD), lambda h: (0, h)),
            pl.BlockSpec((K, 2 * HD), lambda h: (0, h)),
        ],
        out_specs=pl.BlockSpec((Q, 2 * HD), lambda h: (0, h)),
        out_shape=jax.ShapeDtypeStruct((Q, D), jnp.float32),
    )(q4n, k4n, v)

    out = pl.pallas_call(
        _stage_c_kernel,
        out_shape=jax.ShapeDtypeStruct((Q, D), jnp.float32),
    )(merge, qp, q2, p_W, p_b, f_W, f_b, alpha[0])

    return out[:, None, :]
